# baseline (speedup 1.0000x reference)
import functools

import jax
import jax.numpy as jnp
from jax import lax
from jax.experimental import pallas as pl
from jax.experimental.pallas import tpu as pltpu
from jax.experimental.pallas import tpu_sc as plsc

EMB = 64
NC = 2
NS = 16
NW = NC * NS
VOC = 1000000
CPW = 244
NBLK = CPW // 2
TCOLS = CPW * NW
LEFT = 3
VTAIL = VOC - (TCOLS + LEFT) * 128
SROWS = VOC * EMB // 128
TROWS = VTAIL * EMB // 128

BATCH = 16384
HIST = 50
CB = 512


def _mesh():
    return plsc.VectorSubcoreMesh(
        core_axis_name="c", subcore_axis_name="s", num_cores=NC, num_subcores=NS
    )


def _wid():
    return lax.axis_index("s") * NC + lax.axis_index("c")


def _splat(v):
    return jnp.full((16,), v, dtype=jnp.int32)


@functools.partial(
    pl.kernel,
    out_type=jax.ShapeDtypeStruct((SROWS, 128), jnp.float32),
    mesh=_mesh(),
    scratch_types=[
        pltpu.VMEM((64, 256), jnp.float32),
        pltpu.VMEM((64, 256), jnp.float32),
        pltpu.VMEM((128, 128), jnp.float32),
        pltpu.VMEM((128, 128), jnp.float32),
        pltpu.SemaphoreType.DMA,
        pltpu.SemaphoreType.DMA,
        pltpu.SemaphoreType.DMA,
        pltpu.SemaphoreType.DMA,
    ],
    compiler_params=pltpu.CompilerParams(
        use_tc_tiling_on_sc=True, needs_layout_passes=False
    ),
)
def _stage_table(wt_hbm, tail_hbm, st_hbm, ib0, ib1, ob0, ob1, i0, i1, o0, o1):
    wid = _wid()
    ib = (ib0, ib1)
    ob = (ob0, ob1)
    isem = (i0, i1)
    osem = (o0, o1)
    iota = lax.iota(jnp.int32, 16)
    rows4 = [iota + (j * 16) for j in range(4)]
    cw = wid * CPW

    def i_start(j, p):
        pltpu.async_copy(
            wt_hbm.at[:, pl.ds((cw + 2 * j) * 128, 256)], ib[p], isem[p]
        )

    def i_wait(p):
        pltpu.make_async_copy(
            wt_hbm.at[:, pl.ds(0, 256)], ib[p], isem[p]
        ).wait()

    def o_start(j, p):
        pltpu.async_copy(
            ob[p], st_hbm.at[pl.ds((cw + 2 * j) * 64, 128)], osem[p]
        )

    def o_wait(p):
        pltpu.make_async_copy(
            ob[p], st_hbm.at[pl.ds(0, 128)], osem[p]
        ).wait()

    def transpose(p):
        src, dst = ib[p], ob[p]

        @plsc.parallel_loop(0, 128, unroll=4)
        def _(rr):
            vals = [
                plsc.load_gather(
                    src,
                    [rows4[cc0 % 4], _splat(2 * rr + (1 if cc0 >= 4 else 0))],
                )
                for cc0 in range(8)
            ]
            for cc0 in range(8):
                dst[rr, pl.ds(cc0 * 16, 16)] = vals[cc0]

    i_start(0, 0)

    @pl.loop(0, NBLK // 2)
    def _(jh):
        j0 = jh * 2
        i_start(j0 + 1, 1)
        i_wait(0)

        @pl.when(j0 >= 2)
        def _():
            o_wait(0)

        transpose(0)
        o_start(j0, 0)

        @pl.when(j0 != NBLK - 2)
        def _():
            i_start(j0 + 2, 0)

        i_wait(1)

        @pl.when(j0 > 0)
        def _():
            o_wait(1)

        transpose(1)
        o_start(j0 + 1, 1)

    o_wait(0)
    o_wait(1)

    @pl.when(wid < LEFT)
    def _():
        col = TCOLS + wid
        pltpu.sync_copy(
            wt_hbm.at[:, pl.ds(col * 128, 128)], ib0.at[:, pl.ds(0, 128)]
        )

        @plsc.parallel_loop(0, 64, unroll=4)
        def _(rr):
            vals = [
                plsc.load_gather(
                    ib0,
                    [rows4[cc0 % 4], _splat(2 * rr + (1 if cc0 >= 4 else 0))],
                )
                for cc0 in range(8)
            ]
            for cc0 in range(8):
                ob0[rr, pl.ds(cc0 * 16, 16)] = vals[cc0]

        pltpu.sync_copy(ob0.at[pl.ds(0, 64)], st_hbm.at[pl.ds(col * 64, 64)])

    @pl.when(wid == 31)
    def _():
        pltpu.sync_copy(tail_hbm.at[pl.ds(0, 64)], ib1.at[:, pl.ds(0, 128)])
        pltpu.sync_copy(
            ib1.at[pl.ds(0, 64), pl.ds(0, 128)],
            st_hbm.at[pl.ds((TCOLS + LEFT) * 64, 64)],
        )
        pltpu.sync_copy(tail_hbm.at[pl.ds(64, 32)], ob1.at[pl.ds(0, 32)])
        pltpu.sync_copy(
            ob1.at[pl.ds(0, 32)],
            st_hbm.at[pl.ds((TCOLS + LEFT) * 64 + 64, 32)],
        )


@functools.partial(
    pl.kernel,
    out_type=jax.ShapeDtypeStruct((HIST, 8, 128, 8, 128), jnp.float32),
    mesh=_mesh(),
    scratch_types=[
        pltpu.VMEM((CB,), jnp.int32),
        pltpu.VMEM((CB,), jnp.int32),
        pltpu.VMEM((CB, EMB), jnp.float32),
        pltpu.VMEM((CB, EMB), jnp.float32),
        pltpu.VMEM((8, 4, 8, 128), jnp.float32),
        pltpu.SemaphoreType.DMA,
        pltpu.SemaphoreType.DMA,
        pltpu.SemaphoreType.DMA,
        pltpu.SemaphoreType.DMA,
        pltpu.SemaphoreType.DMA,
    ],
    compiler_params=pltpu.CompilerParams(
        use_tc_tiling_on_sc=False, needs_layout_passes=False
    ),
)
def _gather_t(xt_hbm, st_hbm, out_hbm, ibuf0, ibuf1, g0, g1, obuf,
              is0, is1, gs0, gs1, osm):
    wid = _wid()
    ibuf = (ibuf0, ibuf1)
    gbuf = (g0, g1)
    isem = (is0, is1)
    gsem = (gs0, gs1)
    iota = lax.iota(jnp.int32, 16)
    rows32 = [iota + (j * 16) for j in range(32)]
    b0 = wid * CB
    c0 = wid * 4

    def x_start(h, p):
        pltpu.async_copy(xt_hbm.at[h, pl.ds(b0, CB)], ibuf[p], isem[p])

    def x_wait(p):
        pltpu.make_async_copy(
            xt_hbm.at[0, pl.ds(0, CB)], ibuf[p], isem[p]
        ).wait()

    def g_start(p):
        pltpu.async_copy(st_hbm.at[ibuf[p]], gbuf[p], gsem[p])

    def g_wait(p):
        pltpu.make_async_copy(st_hbm.at[ibuf[p]], gbuf[p], gsem[p]).wait()

    def o_drain():
        for _ in range(8):
            pltpu.make_async_copy(
                obuf.at[0], out_hbm.at[0, 0, pl.ds(0, 4)], osm
            ).wait()

    def transpose_and_write(h, p):
        src = gbuf[p]
        for r in range(8):

            @plsc.parallel_loop(0, 8, unroll=4)
            def _(fr):
                cols = _splat(r * 8 + fr)
                vals = [
                    plsc.load_gather(src, [rows32[j], cols]) for j in range(32)
                ]
                for j in range(32):
                    obuf[r, j // 8, fr, pl.ds((j % 8) * 16, 16)] = vals[j]

            pltpu.async_copy(
                obuf.at[r], out_hbm.at[h, r, pl.ds(c0, 4)], osm
            )

    x_start(0, 0)
    x_start(1, 1)
    x_wait(0)
    g_start(0)

    @pl.loop(0, HIST // 2)
    def _(k):
        h0 = k * 2
        x_wait(1)
        g_start(1)
        g_wait(0)

        @pl.when(k != HIST // 2 - 1)
        def _():
            x_start(h0 + 2, 0)

        @pl.when(k > 0)
        def _():
            o_drain()

        transpose_and_write(h0, 0)

        @pl.when(k != HIST // 2 - 1)
        def _():
            x_wait(0)
            g_start(0)

        g_wait(1)

        @pl.when(k != HIST // 2 - 1)
        def _():
            x_start(h0 + 3, 1)

        o_drain()
        transpose_and_write(h0 + 1, 1)

    o_drain()


def kernel(x, emb_weight):
    xt = x.T.astype(jnp.int32)
    tail = emb_weight[(TCOLS + LEFT) * 128 :].reshape(TROWS, 128)
    staged = _stage_table(emb_weight.T, tail)
    staged_lin = staged.reshape(VOC, EMB)
    out5 = _gather_t(xt, staged_lin)
    return (
        out5.transpose(0, 1, 3, 2, 4)
        .reshape(HIST, EMB, BATCH)
        .transpose(2, 0, 1)
    )


# speedup vs baseline: 1.6228x; 1.6228x over previous
import functools

import jax
import jax.numpy as jnp
from jax import lax
from jax.experimental import pallas as pl
from jax.experimental.pallas import tpu as pltpu
from jax.experimental.pallas import tpu_sc as plsc

EMB = 64
NC = 2
NS = 16
NW = NC * NS
CHUNK = 640


def _lookup(idx, table):
    B = idx.shape[0]
    assert B % NW == 0
    bpw = B // NW
    assert bpw % CHUNK == 0 and (bpw // CHUNK) % 2 == 0
    nchunk = bpw // CHUNK

    mesh = plsc.VectorSubcoreMesh(
        core_axis_name="c", subcore_axis_name="s", num_cores=NC, num_subcores=NS
    )

    @functools.partial(
        pl.kernel,
        out_type=jax.ShapeDtypeStruct((B, EMB), jnp.float32),
        mesh=mesh,
        scratch_types=[
            pltpu.VMEM((bpw,), jnp.int32),
            pltpu.VMEM((CHUNK, EMB), jnp.float32),
            pltpu.VMEM((CHUNK, EMB), jnp.float32),
            pltpu.SemaphoreType.DMA,
            pltpu.SemaphoreType.DMA,
            pltpu.SemaphoreType.DMA,
            pltpu.SemaphoreType.DMA,
        ],
        compiler_params=pltpu.CompilerParams(use_tc_tiling_on_sc=False),
    )
    def body(idx_hbm, table_hbm, out_hbm, idx_v, rows0, rows1, g0, g1, o0, o1):
        wid = lax.axis_index("s") * NC + lax.axis_index("c")
        base = wid * bpw
        rows = (rows0, rows1)
        gsem = (g0, g1)
        osem = (o0, o1)

        pltpu.sync_copy(idx_hbm.at[pl.ds(base, bpw)], idx_v)

        def g_start(t, b):
            pltpu.async_copy(
                table_hbm.at[idx_v.at[pl.ds(t * CHUNK, CHUNK)]], rows[b], gsem[b]
            )

        def g_wait(b):
            pltpu.make_async_copy(
                table_hbm.at[idx_v.at[pl.ds(0, CHUNK)]], rows[b], gsem[b]
            ).wait()

        def o_start(t, b):
            pltpu.async_copy(
                rows[b], out_hbm.at[pl.ds(base + t * CHUNK, CHUNK)], osem[b]
            )

        def o_wait(b):
            pltpu.make_async_copy(
                rows[b], out_hbm.at[pl.ds(base, CHUNK)], osem[b]
            ).wait()

        g_start(0, 0)
        g_start(1, 1)
        g_wait(0)
        o_start(0, 0)

        @pl.loop(1, nchunk - 1, step=2)
        def mid(c):
            for b in (1, 0):
                t = c if b == 1 else c + 1
                nb = 1 - b
                o_wait(nb)
                g_start(t + 1, nb)
                g_wait(b)
                o_start(t, b)

        g_wait(1)
        o_start(nchunk - 1, 1)
        o_wait(0)
        o_wait(1)

    return body(idx, table)


def kernel(x, emb_weight):
    b, h = x.shape
    idx = x.reshape(-1).astype(jnp.int32) * 2
    padded = jnp.pad(emb_weight, ((0, 0), (0, 64))).reshape(2000000, EMB)
    out = _lookup(idx, padded)
    return out.reshape(b, h, EMB)
